# baseline (speedup 1.0000x reference)
import functools

import jax
import jax.numpy as jnp
from jax import lax
from jax.experimental import pallas as pl
from jax.experimental.pallas import tpu as pltpu
from jax.experimental.pallas import tpu_sc as plsc

NC = 2
NS = 16
NW = NC * NS
LANES = 16
CHUNK = 640
CROWS = CHUNK // 128
PKROWS = -(-3 * CROWS // 8) * 8


@functools.lru_cache(maxsize=None)
def _make_agg(n_nodes, d, nchunks):
    rows_per_tile = n_nodes // NS
    mesh = plsc.VectorSubcoreMesh(
        core_axis_name="c", subcore_axis_name="s", num_cores=NC, num_subcores=NS
    )

    np8 = n_nodes * d // 128
    rpr = rows_per_tile
    for cand in range(368, 7, -8):
        if rows_per_tile % cand == 0 and (cand * d) % 128 == 0:
            rpr = cand
            break

    @functools.partial(
        pl.kernel,
        out_type=jax.ShapeDtypeStruct((NC, np8, 128), jnp.int32),
        mesh=mesh,
        scratch_types=[
            pltpu.VMEM_SHARED((n_nodes, d), jnp.float32),
            pltpu.VMEM((2, PKROWS, 128), jnp.int32),
            pltpu.VMEM((2, CHUNK, d), jnp.float32),
            pltpu.VMEM((368 * 16 // 128, 128), jnp.int32),
            pltpu.SemaphoreType.DMA,
            pltpu.SemaphoreType.DMA,
            pltpu.SemaphoreType.DMA,
            pltpu.SemaphoreType.DMA,
        ],
        compiler_params=pltpu.CompilerParams(
            use_tc_tiling_on_sc=False, needs_layout_passes=False
        ),
    )
    def agg(zeros_hbm, xw_hbm, pk_hbm, out_hbm,
            acc, stg_v, msg_v, rb_v, gsem0, gsem1, ssem0, ssem1):
        cid = lax.axis_index("c")
        sid = lax.axis_index("s")
        wid = sid * NC + cid
        gsem = (gsem0, gsem1)
        ssem = (ssem0, ssem1)
        xwt = xw_hbm

        pltpu.sync_copy(
            zeros_hbm.at[pl.ds(sid * rows_per_tile, rows_per_tile)],
            acc.at[pl.ds(sid * rows_per_tile, rows_per_tile)],
        )
        plsc.subcore_barrier()

        def fire_stage(i, b):
            ci = wid * nchunks + i
            pltpu.async_copy(pk_hbm.at[ci], stg_v.at[b], gsem[b])

        def fire_gather(i, b):
            ci = wid * nchunks + i
            pltpu.make_async_copy(pk_hbm.at[ci], stg_v.at[b], gsem[b]).wait()
            for j in range(CROWS):
                pltpu.async_copy(
                    xwt.at[stg_v.at[b, j]],
                    msg_v.at[b, pl.ds(j * 128, 128)],
                    gsem[b],
                )

        def wait_gather(b):
            pltpu.make_async_copy(
                xwt.at[pl.ds(0, CHUNK)], msg_v.at[b], gsem[b]
            ).wait()

        def fire_scatter(b):
            for j in range(CROWS):
                pltpu.async_copy(
                    msg_v.at[b, pl.ds(j * 128, 128)],
                    acc.at[stg_v.at[b, CROWS + j]],
                    ssem[b],
                    add=True,
                )

        def drain_scatter(b):
            pltpu.make_async_copy(
                xwt.at[pl.ds(0, CHUNK)], msg_v.at[b], ssem[b]
            ).wait()

        def scale(b):
            @plsc.parallel_loop(0, CHUNK // LANES, unroll=2)
            def scale_body(k):
                w16i = stg_v[b, 2 * CROWS + k // 8, pl.ds((k % 8) * LANES, LANES)]
                w16 = plsc.bitcast(w16i, jnp.float32)
                base = k * LANES
                for j in range(LANES):
                    wj = lax.gather(
                        w16,
                        jnp.full((LANES, 1), j, jnp.int32),
                        lax.GatherDimensionNumbers(
                            offset_dims=(),
                            collapsed_slice_dims=(0,),
                            start_index_map=(0,),
                        ),
                        (1,),
                        mode=lax.GatherScatterMode.PROMISE_IN_BOUNDS,
                    )
                    msg_v[b, base + j, :] = msg_v[b, base + j, :] * wj

        def step(a, b):
            ob = 1 - b

            @pl.when(a > 0)
            def _():
                drain_scatter(ob)

            @pl.when(a + 1 < nchunks)
            def _():
                fire_stage(a + 1, ob)

            wait_gather(b)

            @pl.when(a + 1 < nchunks)
            def _():
                fire_gather(a + 1, ob)

            scale(b)
            fire_scatter(b)

        fire_stage(0, 0)
        fire_gather(0, 0)

        def pair_body(t, carry):
            step(2 * t, 0)

            @pl.when(2 * t + 1 < nchunks)
            def _():
                step(2 * t + 1, 1)

            return carry

        lax.fori_loop(0, (nchunks + 1) // 2, pair_body, 0)
        drain_scatter((nchunks - 1) % 2)
        plsc.subcore_barrier()

        ngroups = rows_per_tile // rpr
        orows = rpr * d // 128
        obase = sid * (rows_per_tile * d // 128)

        def rd_body(g, carry):
            pltpu.sync_copy(
                acc.at[pl.ds(sid * rows_per_tile + g * rpr, rpr)],
                msg_v.at[0, pl.ds(0, rpr)],
            )

            def rp_body(k, c2):
                v = msg_v[0, k, :]
                rb_v[k // 8, pl.ds((k % 8) * LANES, LANES)] = plsc.bitcast(
                    v, jnp.int32
                )
                return c2

            lax.fori_loop(0, rpr, rp_body, 0)
            pltpu.sync_copy(
                rb_v.at[pl.ds(0, orows)],
                out_hbm.at[cid, pl.ds(obase + g * orows, orows)],
            )
            return carry

        lax.fori_loop(0, ngroups, rd_body, 0)

    return agg


def _pick_blk(n):
    for b in range(2048, 7, -1):
        if n % b == 0 and b % 8 == 0:
            return b
    return n


def _mm_body(x_ref, w_ref, o_ref):
    o_ref[...] = jnp.dot(x_ref[...], w_ref[...], preferred_element_type=jnp.float32)


def _matmul(x, w):
    n, k = x.shape
    m = w.shape[1]
    blk = _pick_blk(n)
    return pl.pallas_call(
        _mm_body,
        grid=(n // blk,),
        in_specs=[
            pl.BlockSpec((blk, k), lambda i: (i, 0)),
            pl.BlockSpec((k, m), lambda i: (0, 0)),
        ],
        out_specs=pl.BlockSpec((blk, m), lambda i: (i, 0)),
        out_shape=jax.ShapeDtypeStruct((n, m), jnp.float32),
    )(x, w)


def _relu_mm_body(p_ref, b_ref, w_ref, o_ref):
    h = jnp.maximum(p_ref[0] + p_ref[1] + b_ref[...], 0.0)
    o_ref[...] = jnp.dot(h, w_ref[...], preferred_element_type=jnp.float32)


def _relu_matmul_packed(p, b, wbd):
    _, n8, d8 = p.shape
    m8 = wbd.shape[1]
    blk = _pick_blk(n8)
    return pl.pallas_call(
        _relu_mm_body,
        grid=(n8 // blk,),
        in_specs=[
            pl.BlockSpec((2, blk, d8), lambda i: (0, i, 0)),
            pl.BlockSpec((1, d8), lambda i: (0, 0)),
            pl.BlockSpec((d8, m8), lambda i: (0, 0)),
        ],
        out_specs=pl.BlockSpec((blk, m8), lambda i: (i, 0)),
        out_shape=jax.ShapeDtypeStruct((n8, m8), jnp.float32),
    )(p, b.reshape(1, d8), wbd)


def _relu_body(p_ref, b_ref, o_ref):
    o_ref[...] = jnp.maximum(p_ref[0] + p_ref[1] + b_ref[...], 0.0)


def _relu_bias_packed(p, b):
    _, n8, d8 = p.shape
    blk = _pick_blk(n8)
    return pl.pallas_call(
        _relu_body,
        grid=(n8 // blk,),
        in_specs=[
            pl.BlockSpec((2, blk, d8), lambda i: (0, i, 0)),
            pl.BlockSpec((1, d8), lambda i: (0, 0)),
        ],
        out_specs=pl.BlockSpec((blk, d8), lambda i: (i, 0)),
        out_shape=jax.ShapeDtypeStruct((n8, d8), jnp.float32),
    )(p, b.reshape(1, d8))


def _final_body(p_ref, w_ref, b_ref, o_ref):
    z = jnp.dot(p_ref[0] + p_ref[1], w_ref[...], preferred_element_type=jnp.float32)
    z = z + b_ref[...]
    m = z.shape[1] // 8
    parts = []
    for j in range(8):
        zj = z[:, j * m:(j + 1) * m]
        mj = jnp.max(zj, axis=1, keepdims=True)
        ej = jnp.exp(zj - mj)
        sj = jnp.sum(ej, axis=1, keepdims=True)
        parts.append(zj - mj - jnp.log(sj))
    o_ref[...] = jnp.concatenate(parts, axis=1)


def _final_packed(p, wbd, b):
    _, n8, d8 = p.shape
    m8 = wbd.shape[1]
    blk = _pick_blk(n8)
    return pl.pallas_call(
        _final_body,
        grid=(n8 // blk,),
        in_specs=[
            pl.BlockSpec((2, blk, d8), lambda i: (0, i, 0)),
            pl.BlockSpec((d8, m8), lambda i: (0, 0)),
            pl.BlockSpec((1, m8), lambda i: (0, 0)),
        ],
        out_specs=pl.BlockSpec((blk, m8), lambda i: (i, 0)),
        out_shape=jax.ShapeDtypeStruct((n8, m8), jnp.float32),
    )(p, wbd, jnp.tile(b, 8).reshape(1, m8))


def kernel(edge_index, features, edge_weights, W0, b0, W1, b1, W2, b2):
    n_nodes, _ = features.shape
    n_edges = edge_index.shape[1]
    d = W0.shape[1]

    n_nodes_p = -(-n_nodes // (NS * 8)) * (NS * 8)

    per_w = -(-n_edges // (NW * CHUNK)) * CHUNK
    nchunks = per_w // CHUNK
    n_pad = NW * per_w
    pad = n_pad - n_edges

    row = jnp.concatenate([edge_index[0], jnp.zeros((pad,), jnp.int32)])
    col = jnp.concatenate([edge_index[1], jnp.zeros((pad,), jnp.int32)])
    wts = jnp.concatenate([edge_weights, jnp.zeros((pad,), jnp.float32)])
    ntot = n_pad // CHUNK
    row3 = row.reshape(ntot, CROWS, 128)
    col3 = col.reshape(ntot, CROWS, 128)
    wts3 = lax.bitcast_convert_type(wts, jnp.int32).reshape(ntot, CROWS, 128)
    zpad = jnp.zeros((ntot, PKROWS - 3 * CROWS, 128), jnp.int32)
    packed = jnp.concatenate([row3, col3, wts3, zpad], axis=1)
    np8 = n_nodes_p // 8
    zeros = jnp.zeros((n_nodes_p, d), jnp.float32)

    agg = _make_agg(n_nodes_p, d, nchunks)
    bd1 = jnp.kron(jnp.eye(8, dtype=jnp.float32), W1)
    bd2 = jnp.kron(jnp.eye(8, dtype=jnp.float32), W2)

    xw0 = _matmul(features, W0)
    xw0 = jnp.pad(xw0, ((0, n_nodes_p - n_nodes), (0, 0)))
    p0 = agg(zeros, xw0, packed)
    pp0 = lax.bitcast_convert_type(p0, jnp.float32)
    xw1 = _relu_matmul_packed(pp0, jnp.tile(b0, 8), bd1)
    p1 = agg(zeros, xw1.reshape(n_nodes_p, d), packed)
    pp1 = lax.bitcast_convert_type(p1, jnp.float32)
    h1 = _relu_bias_packed(pp1, jnp.tile(b1, 8))
    p2 = agg(zeros, h1.reshape(n_nodes_p, d), packed)
    pp2 = lax.bitcast_convert_type(p2, jnp.float32)
    outp = _final_packed(pp2, bd2, b2)
    out = outp.reshape(n_nodes_p, 64)
    return out[:n_nodes]


# speedup vs baseline: 1.0405x; 1.0405x over previous
import functools

import jax
import jax.numpy as jnp
from jax import lax
from jax.experimental import pallas as pl
from jax.experimental.pallas import tpu as pltpu
from jax.experimental.pallas import tpu_sc as plsc

NC = 2
NS = 16
NW = NC * NS
LANES = 16
CHUNK = 640
CROWS = CHUNK // 128
PKROWS = -(-3 * CROWS // 8) * 8


@functools.lru_cache(maxsize=None)
def _make_agg(n_nodes, d, nchunks):
    rows_per_tile = n_nodes // NS
    mesh = plsc.VectorSubcoreMesh(
        core_axis_name="c", subcore_axis_name="s", num_cores=NC, num_subcores=NS
    )

    np8 = n_nodes * d // 128
    rpr = rows_per_tile
    for cand in range(368, 7, -8):
        if rows_per_tile % cand == 0 and (cand * d) % 128 == 0:
            rpr = cand
            break

    @functools.partial(
        pl.kernel,
        out_type=jax.ShapeDtypeStruct((NC, np8, 128), jnp.int32),
        mesh=mesh,
        scratch_types=[
            pltpu.VMEM_SHARED((n_nodes, d), jnp.float32),
            pltpu.VMEM((2, PKROWS, 128), jnp.int32),
            pltpu.VMEM((2, CHUNK, d), jnp.float32),
            pltpu.VMEM((368 * 16 // 128, 128), jnp.int32),
            pltpu.SemaphoreType.DMA,
            pltpu.SemaphoreType.DMA,
            pltpu.SemaphoreType.DMA,
            pltpu.SemaphoreType.DMA,
        ],
        compiler_params=pltpu.CompilerParams(
            use_tc_tiling_on_sc=False, needs_layout_passes=False
        ),
    )
    def agg(xw_hbm, pk_hbm, out_hbm,
            acc, stg_v, msg_v, rb_v, gsem0, gsem1, ssem0, ssem1):
        cid = lax.axis_index("c")
        sid = lax.axis_index("s")
        wid = sid * NC + cid
        gsem = (gsem0, gsem1)
        ssem = (ssem0, ssem1)
        xwt = xw_hbm

        def z_body(k, c2):
            msg_v[0, k, :] = jnp.zeros((LANES,), jnp.float32)
            return c2

        lax.fori_loop(0, rpr, z_body, 0)
        for g in range(rows_per_tile // rpr):
            pltpu.sync_copy(
                msg_v.at[0, pl.ds(0, rpr)],
                acc.at[pl.ds(sid * rows_per_tile + g * rpr, rpr)],
            )
        plsc.subcore_barrier()

        def fire_stage(i, b):
            ci = wid * nchunks + i
            pltpu.async_copy(pk_hbm.at[ci], stg_v.at[b], gsem[b])

        def fire_gather(i, b):
            ci = wid * nchunks + i
            pltpu.make_async_copy(pk_hbm.at[ci], stg_v.at[b], gsem[b]).wait()
            for j in range(CROWS):
                pltpu.async_copy(
                    xwt.at[stg_v.at[b, j]],
                    msg_v.at[b, pl.ds(j * 128, 128)],
                    gsem[b],
                )

        def wait_gather(b):
            pltpu.make_async_copy(
                xwt.at[pl.ds(0, CHUNK)], msg_v.at[b], gsem[b]
            ).wait()

        def fire_scatter(b):
            for j in range(CROWS):
                pltpu.async_copy(
                    msg_v.at[b, pl.ds(j * 128, 128)],
                    acc.at[stg_v.at[b, CROWS + j]],
                    ssem[b],
                    add=True,
                )

        def drain_scatter(b):
            pltpu.make_async_copy(
                xwt.at[pl.ds(0, CHUNK)], msg_v.at[b], ssem[b]
            ).wait()

        def scale(b):
            @plsc.parallel_loop(0, CHUNK // LANES, unroll=2)
            def scale_body(k):
                w16i = stg_v[b, 2 * CROWS + k // 8, pl.ds((k % 8) * LANES, LANES)]
                w16 = plsc.bitcast(w16i, jnp.float32)
                base = k * LANES
                for j in range(LANES):
                    wj = lax.gather(
                        w16,
                        jnp.full((LANES, 1), j, jnp.int32),
                        lax.GatherDimensionNumbers(
                            offset_dims=(),
                            collapsed_slice_dims=(0,),
                            start_index_map=(0,),
                        ),
                        (1,),
                        mode=lax.GatherScatterMode.PROMISE_IN_BOUNDS,
                    )
                    msg_v[b, base + j, :] = msg_v[b, base + j, :] * wj

        def step(a, b):
            ob = 1 - b

            @pl.when(a > 0)
            def _():
                drain_scatter(ob)

            @pl.when(a + 1 < nchunks)
            def _():
                fire_stage(a + 1, ob)

            wait_gather(b)

            @pl.when(a + 1 < nchunks)
            def _():
                fire_gather(a + 1, ob)

            scale(b)
            fire_scatter(b)

        fire_stage(0, 0)
        fire_gather(0, 0)

        def pair_body(t, carry):
            step(2 * t, 0)

            @pl.when(2 * t + 1 < nchunks)
            def _():
                step(2 * t + 1, 1)

            return carry

        lax.fori_loop(0, (nchunks + 1) // 2, pair_body, 0)
        drain_scatter((nchunks - 1) % 2)
        plsc.subcore_barrier()

        ngroups = rows_per_tile // rpr
        orows = rpr * d // 128
        obase = sid * (rows_per_tile * d // 128)

        def rd_fire(g, mb):
            pltpu.async_copy(
                acc.at[pl.ds(sid * rows_per_tile + g * rpr, rpr)],
                msg_v.at[mb, pl.ds(0, rpr)],
                gsem[mb],
            )

        def rd_wait(g, mb):
            pltpu.make_async_copy(
                acc.at[pl.ds(sid * rows_per_tile + g * rpr, rpr)],
                msg_v.at[mb, pl.ds(0, rpr)],
                gsem[mb],
            ).wait()

        rd_fire(0, 0)
        for g in range(ngroups):
            mb = g % 2
            rd_wait(g, mb)
            if g + 1 < ngroups:
                rd_fire(g + 1, 1 - mb)

            @plsc.parallel_loop(0, rpr, unroll=2)
            def rp_body(k):
                v = msg_v[mb, k, :]
                rb_v[k // 8, pl.ds((k % 8) * LANES, LANES)] = plsc.bitcast(
                    v, jnp.int32
                )

            pltpu.sync_copy(
                rb_v.at[pl.ds(0, orows)],
                out_hbm.at[cid, pl.ds(obase + g * orows, orows)],
            )

    return agg


def _pick_blk(n):
    for b in range(2048, 7, -1):
        if n % b == 0 and b % 8 == 0:
            return b
    return n


def _mm_body(x_ref, w_ref, o_ref):
    o_ref[...] = jnp.dot(x_ref[...], w_ref[...], preferred_element_type=jnp.float32)


def _matmul(x, w):
    n, k = x.shape
    m = w.shape[1]
    blk = _pick_blk(n)
    return pl.pallas_call(
        _mm_body,
        grid=(n // blk,),
        in_specs=[
            pl.BlockSpec((blk, k), lambda i: (i, 0)),
            pl.BlockSpec((k, m), lambda i: (0, 0)),
        ],
        out_specs=pl.BlockSpec((blk, m), lambda i: (i, 0)),
        out_shape=jax.ShapeDtypeStruct((n, m), jnp.float32),
    )(x, w)


def _relu_mm_body(p_ref, b_ref, w_ref, o_ref):
    h = jnp.maximum(p_ref[0] + p_ref[1] + b_ref[...], 0.0)
    o_ref[...] = jnp.dot(h, w_ref[...], preferred_element_type=jnp.float32)


def _relu_matmul_packed(p, b, wbd):
    _, n8, d8 = p.shape
    m8 = wbd.shape[1]
    blk = _pick_blk(n8)
    return pl.pallas_call(
        _relu_mm_body,
        grid=(n8 // blk,),
        in_specs=[
            pl.BlockSpec((2, blk, d8), lambda i: (0, i, 0)),
            pl.BlockSpec((1, d8), lambda i: (0, 0)),
            pl.BlockSpec((d8, m8), lambda i: (0, 0)),
        ],
        out_specs=pl.BlockSpec((blk, m8), lambda i: (i, 0)),
        out_shape=jax.ShapeDtypeStruct((n8, m8), jnp.float32),
    )(p, b.reshape(1, d8), wbd)


def _relu_body(p_ref, b_ref, o_ref):
    o_ref[...] = jnp.maximum(p_ref[0] + p_ref[1] + b_ref[...], 0.0)


def _relu_bias_packed(p, b):
    _, n8, d8 = p.shape
    blk = _pick_blk(n8)
    return pl.pallas_call(
        _relu_body,
        grid=(n8 // blk,),
        in_specs=[
            pl.BlockSpec((2, blk, d8), lambda i: (0, i, 0)),
            pl.BlockSpec((1, d8), lambda i: (0, 0)),
        ],
        out_specs=pl.BlockSpec((blk, d8), lambda i: (i, 0)),
        out_shape=jax.ShapeDtypeStruct((n8, d8), jnp.float32),
    )(p, b.reshape(1, d8))


def _final_body(p_ref, w_ref, b_ref, o_ref):
    z = jnp.dot(p_ref[0] + p_ref[1], w_ref[...], preferred_element_type=jnp.float32)
    z = z + b_ref[...]
    m = z.shape[1] // 8
    parts = []
    for j in range(8):
        zj = z[:, j * m:(j + 1) * m]
        mj = jnp.max(zj, axis=1, keepdims=True)
        ej = jnp.exp(zj - mj)
        sj = jnp.sum(ej, axis=1, keepdims=True)
        parts.append(zj - mj - jnp.log(sj))
    o_ref[...] = jnp.concatenate(parts, axis=1)


def _final_packed(p, wbd, b):
    _, n8, d8 = p.shape
    m8 = wbd.shape[1]
    blk = _pick_blk(n8)
    return pl.pallas_call(
        _final_body,
        grid=(n8 // blk,),
        in_specs=[
            pl.BlockSpec((2, blk, d8), lambda i: (0, i, 0)),
            pl.BlockSpec((d8, m8), lambda i: (0, 0)),
            pl.BlockSpec((1, m8), lambda i: (0, 0)),
        ],
        out_specs=pl.BlockSpec((blk, m8), lambda i: (i, 0)),
        out_shape=jax.ShapeDtypeStruct((n8, m8), jnp.float32),
    )(p, wbd, jnp.tile(b, 8).reshape(1, m8))


def kernel(edge_index, features, edge_weights, W0, b0, W1, b1, W2, b2):
    n_nodes, _ = features.shape
    n_edges = edge_index.shape[1]
    d = W0.shape[1]

    n_nodes_p = -(-n_nodes // (NS * 8)) * (NS * 8)

    per_w = -(-n_edges // (NW * CHUNK)) * CHUNK
    nchunks = per_w // CHUNK
    n_pad = NW * per_w
    pad = n_pad - n_edges

    row = jnp.concatenate([edge_index[0], jnp.zeros((pad,), jnp.int32)])
    col = jnp.concatenate([edge_index[1], jnp.zeros((pad,), jnp.int32)])
    wts = jnp.concatenate([edge_weights, jnp.zeros((pad,), jnp.float32)])
    ntot = n_pad // CHUNK
    row3 = row.reshape(ntot, CROWS, 128)
    col3 = col.reshape(ntot, CROWS, 128)
    wts3 = lax.bitcast_convert_type(wts, jnp.int32).reshape(ntot, CROWS, 128)
    zpad = jnp.zeros((ntot, PKROWS - 3 * CROWS, 128), jnp.int32)
    packed = jnp.concatenate([row3, col3, wts3, zpad], axis=1)
    np8 = n_nodes_p // 8

    agg = _make_agg(n_nodes_p, d, nchunks)
    bd1 = jnp.kron(jnp.eye(8, dtype=jnp.float32), W1)
    bd2 = jnp.kron(jnp.eye(8, dtype=jnp.float32), W2)

    xw0 = _matmul(features, W0)
    xw0 = jnp.pad(xw0, ((0, n_nodes_p - n_nodes), (0, 0)))
    p0 = agg(xw0, packed)
    pp0 = lax.bitcast_convert_type(p0, jnp.float32)
    xw1 = _relu_matmul_packed(pp0, jnp.tile(b0, 8), bd1)
    p1 = agg(xw1.reshape(n_nodes_p, d), packed)
    pp1 = lax.bitcast_convert_type(p1, jnp.float32)
    h1 = _relu_bias_packed(pp1, jnp.tile(b1, 8))
    p2 = agg(h1.reshape(n_nodes_p, d), packed)
    pp2 = lax.bitcast_convert_type(p2, jnp.float32)
    outp = _final_packed(pp2, bd2, b2)
    out = outp.reshape(n_nodes_p, 64)
    return out[:n_nodes]


# speedup vs baseline: 1.0961x; 1.0535x over previous
import functools

import jax
import jax.numpy as jnp
from jax import lax
from jax.experimental import pallas as pl
from jax.experimental.pallas import tpu as pltpu
from jax.experimental.pallas import tpu_sc as plsc

NC = 2
NS = 16
NW = NC * NS
LANES = 16
CHUNK = 640
CROWS = CHUNK // 128
PKROWS = -(-3 * CROWS // 8) * 8


@functools.lru_cache(maxsize=None)
def _make_agg(n_nodes, d, f0, f1):
    rows_per_tile = n_nodes // NS
    mesh = plsc.VectorSubcoreMesh(
        core_axis_name="c", subcore_axis_name="s", num_cores=NC, num_subcores=NS
    )

    np8 = n_nodes * d // 128
    rpr = rows_per_tile
    for cand in range(368, 7, -8):
        if rows_per_tile % cand == 0 and (cand * d) % 128 == 0:
            rpr = cand
            break

    @functools.partial(
        pl.kernel,
        out_type=jax.ShapeDtypeStruct((NC, np8, 128), jnp.int32),
        mesh=mesh,
        scratch_types=[
            pltpu.VMEM_SHARED((n_nodes, d), jnp.float32),
            pltpu.VMEM((2, PKROWS, 128), jnp.int32),
            pltpu.VMEM((2, CHUNK, d), jnp.float32),
            pltpu.VMEM((368 * 16 // 128, 128), jnp.int32),
            pltpu.SemaphoreType.DMA,
            pltpu.SemaphoreType.DMA,
            pltpu.SemaphoreType.DMA,
            pltpu.SemaphoreType.DMA,
        ],
        compiler_params=pltpu.CompilerParams(
            use_tc_tiling_on_sc=False, needs_layout_passes=False
        ),
    )
    def agg(xw_hbm, pk_hbm, out_hbm,
            acc, stg_v, msg_v, rb_v, gsem0, gsem1, ssem0, ssem1):
        cid = lax.axis_index("c")
        sid = lax.axis_index("s")
        gsem = (gsem0, gsem1)
        ssem = (ssem0, ssem1)
        n_c = jnp.where(cid == 0, f0, f1)
        cbase = jnp.where(cid == 0, sid * f0, NS * f0 + sid * f1)
        xwt = xw_hbm

        def z_body(k, c2):
            msg_v[0, k, :] = jnp.zeros((LANES,), jnp.float32)
            return c2

        lax.fori_loop(0, rpr, z_body, 0)
        for g in range(rows_per_tile // rpr):
            pltpu.sync_copy(
                msg_v.at[0, pl.ds(0, rpr)],
                acc.at[pl.ds(sid * rows_per_tile + g * rpr, rpr)],
            )
        plsc.subcore_barrier()

        def fire_stage(i, b):
            ci = cbase + i
            pltpu.async_copy(pk_hbm.at[ci], stg_v.at[b], gsem[b])

        def fire_gather(i, b):
            ci = cbase + i
            pltpu.make_async_copy(pk_hbm.at[ci], stg_v.at[b], gsem[b]).wait()
            for j in range(CROWS):
                pltpu.async_copy(
                    xwt.at[stg_v.at[b, j]],
                    msg_v.at[b, pl.ds(j * 128, 128)],
                    gsem[b],
                )

        def wait_gather(b):
            pltpu.make_async_copy(
                xwt.at[pl.ds(0, CHUNK)], msg_v.at[b], gsem[b]
            ).wait()

        def fire_scatter(b):
            for j in range(CROWS):
                pltpu.async_copy(
                    msg_v.at[b, pl.ds(j * 128, 128)],
                    acc.at[stg_v.at[b, CROWS + j]],
                    ssem[b],
                    add=True,
                )

        def drain_scatter(b):
            pltpu.make_async_copy(
                xwt.at[pl.ds(0, CHUNK)], msg_v.at[b], ssem[b]
            ).wait()

        def scale(b):
            @plsc.parallel_loop(0, CHUNK // LANES, unroll=2)
            def scale_body(k):
                w16i = stg_v[b, 2 * CROWS + k // 8, pl.ds((k % 8) * LANES, LANES)]
                w16 = plsc.bitcast(w16i, jnp.float32)
                base = k * LANES
                for j in range(LANES):
                    wj = lax.gather(
                        w16,
                        jnp.full((LANES, 1), j, jnp.int32),
                        lax.GatherDimensionNumbers(
                            offset_dims=(),
                            collapsed_slice_dims=(0,),
                            start_index_map=(0,),
                        ),
                        (1,),
                        mode=lax.GatherScatterMode.PROMISE_IN_BOUNDS,
                    )
                    msg_v[b, base + j, :] = msg_v[b, base + j, :] * wj

        def step(a, b):
            ob = 1 - b

            @pl.when(a > 0)
            def _():
                drain_scatter(ob)

            @pl.when(a + 1 < n_c)
            def _():
                fire_stage(a + 1, ob)

            wait_gather(b)

            @pl.when(a + 1 < n_c)
            def _():
                fire_gather(a + 1, ob)

            scale(b)
            fire_scatter(b)

        fire_stage(0, 0)
        fire_gather(0, 0)

        def pair_body(t, carry):
            @pl.when(2 * t < n_c)
            def _():
                step(2 * t, 0)

            @pl.when(2 * t + 1 < n_c)
            def _():
                step(2 * t + 1, 1)

            return carry

        lax.fori_loop(0, (max(f0, f1) + 1) // 2, pair_body, 0)

        @pl.when(n_c % 2 == 1)
        def _():
            drain_scatter(0)

        @pl.when(n_c % 2 == 0)
        def _():
            drain_scatter(1)

        plsc.subcore_barrier()

        ngroups = rows_per_tile // rpr
        orows = rpr * d // 128
        obase = sid * (rows_per_tile * d // 128)

        def rd_fire(g, mb):
            pltpu.async_copy(
                acc.at[pl.ds(sid * rows_per_tile + g * rpr, rpr)],
                msg_v.at[mb, pl.ds(0, rpr)],
                gsem[mb],
            )

        def rd_wait(g, mb):
            pltpu.make_async_copy(
                acc.at[pl.ds(sid * rows_per_tile + g * rpr, rpr)],
                msg_v.at[mb, pl.ds(0, rpr)],
                gsem[mb],
            ).wait()

        rd_fire(0, 0)
        for g in range(ngroups):
            mb = g % 2
            rd_wait(g, mb)
            if g + 1 < ngroups:
                rd_fire(g + 1, 1 - mb)

            @plsc.parallel_loop(0, rpr, unroll=2)
            def rp_body(k):
                v = msg_v[mb, k, :]
                rb_v[k // 8, pl.ds((k % 8) * LANES, LANES)] = plsc.bitcast(
                    v, jnp.int32
                )

            pltpu.sync_copy(
                rb_v.at[pl.ds(0, orows)],
                out_hbm.at[cid, pl.ds(obase + g * orows, orows)],
            )

    return agg


def _pick_blk(n):
    for b in range(2048, 7, -1):
        if n % b == 0 and b % 8 == 0:
            return b
    return n


def _mm_body(x_ref, w_ref, o_ref):
    o_ref[...] = jnp.dot(x_ref[...], w_ref[...], preferred_element_type=jnp.float32)


def _matmul(x, w):
    n, k = x.shape
    m = w.shape[1]
    blk = _pick_blk(n)
    return pl.pallas_call(
        _mm_body,
        grid=(n // blk,),
        in_specs=[
            pl.BlockSpec((blk, k), lambda i: (i, 0)),
            pl.BlockSpec((k, m), lambda i: (0, 0)),
        ],
        out_specs=pl.BlockSpec((blk, m), lambda i: (i, 0)),
        out_shape=jax.ShapeDtypeStruct((n, m), jnp.float32),
    )(x, w)


def _relu_mm_body(p_ref, b_ref, w_ref, o_ref):
    h = jnp.maximum(p_ref[0] + p_ref[1] + b_ref[...], 0.0)
    o_ref[...] = jnp.dot(h, w_ref[...], preferred_element_type=jnp.float32)


def _relu_matmul_packed(p, b, wbd):
    _, n8, d8 = p.shape
    m8 = wbd.shape[1]
    blk = _pick_blk(n8)
    return pl.pallas_call(
        _relu_mm_body,
        grid=(n8 // blk,),
        in_specs=[
            pl.BlockSpec((2, blk, d8), lambda i: (0, i, 0)),
            pl.BlockSpec((1, d8), lambda i: (0, 0)),
            pl.BlockSpec((d8, m8), lambda i: (0, 0)),
        ],
        out_specs=pl.BlockSpec((blk, m8), lambda i: (i, 0)),
        out_shape=jax.ShapeDtypeStruct((n8, m8), jnp.float32),
    )(p, b.reshape(1, d8), wbd)


def _relu_body(p_ref, b_ref, o_ref):
    o_ref[...] = jnp.maximum(p_ref[0] + p_ref[1] + b_ref[...], 0.0)


def _relu_bias_packed(p, b):
    _, n8, d8 = p.shape
    blk = _pick_blk(n8)
    return pl.pallas_call(
        _relu_body,
        grid=(n8 // blk,),
        in_specs=[
            pl.BlockSpec((2, blk, d8), lambda i: (0, i, 0)),
            pl.BlockSpec((1, d8), lambda i: (0, 0)),
        ],
        out_specs=pl.BlockSpec((blk, d8), lambda i: (i, 0)),
        out_shape=jax.ShapeDtypeStruct((n8, d8), jnp.float32),
    )(p, b.reshape(1, d8))


def _final_body(p_ref, w_ref, b_ref, o_ref):
    z = jnp.dot(p_ref[0] + p_ref[1], w_ref[...], preferred_element_type=jnp.float32)
    z = z + b_ref[...]
    m = z.shape[1] // 8
    parts = []
    for j in range(8):
        zj = z[:, j * m:(j + 1) * m]
        mj = jnp.max(zj, axis=1, keepdims=True)
        ej = jnp.exp(zj - mj)
        sj = jnp.sum(ej, axis=1, keepdims=True)
        parts.append(zj - mj - jnp.log(sj))
    o_ref[...] = jnp.concatenate(parts, axis=1)


def _final_packed(p, wbd, b):
    _, n8, d8 = p.shape
    m8 = wbd.shape[1]
    blk = _pick_blk(n8)
    return pl.pallas_call(
        _final_body,
        grid=(n8 // blk,),
        in_specs=[
            pl.BlockSpec((2, blk, d8), lambda i: (0, i, 0)),
            pl.BlockSpec((d8, m8), lambda i: (0, 0)),
            pl.BlockSpec((1, m8), lambda i: (0, 0)),
        ],
        out_specs=pl.BlockSpec((blk, m8), lambda i: (i, 0)),
        out_shape=jax.ShapeDtypeStruct((n8, m8), jnp.float32),
    )(p, wbd, jnp.tile(b, 8).reshape(1, m8))


def kernel(edge_index, features, edge_weights, W0, b0, W1, b1, W2, b2):
    n_nodes, _ = features.shape
    n_edges = edge_index.shape[1]
    d = W0.shape[1]

    n_nodes_p = -(-n_nodes // (NS * 8)) * (NS * 8)

    per_w = -(-n_edges // (NW * CHUNK)) * CHUNK
    nchunks = per_w // CHUNK
    n_pad = NW * per_w
    pad = n_pad - n_edges
    f0 = int(round(2 * nchunks * 0.56))
    f1 = 2 * nchunks - f0

    row = jnp.concatenate([edge_index[0], jnp.zeros((pad,), jnp.int32)])
    col = jnp.concatenate([edge_index[1], jnp.zeros((pad,), jnp.int32)])
    wts = jnp.concatenate([edge_weights, jnp.zeros((pad,), jnp.float32)])
    ntot = n_pad // CHUNK
    row3 = row.reshape(ntot, CROWS, 128)
    col3 = col.reshape(ntot, CROWS, 128)
    wts3 = lax.bitcast_convert_type(wts, jnp.int32).reshape(ntot, CROWS, 128)
    zpad = jnp.zeros((ntot, PKROWS - 3 * CROWS, 128), jnp.int32)
    packed = jnp.concatenate([row3, col3, wts3, zpad], axis=1)
    np8 = n_nodes_p // 8

    agg = _make_agg(n_nodes_p, d, f0, f1)
    bd1 = jnp.kron(jnp.eye(8, dtype=jnp.float32), W1)
    bd2 = jnp.kron(jnp.eye(8, dtype=jnp.float32), W2)

    xw0 = _matmul(features, W0)
    xw0 = jnp.pad(xw0, ((0, n_nodes_p - n_nodes), (0, 0)))
    p0 = agg(xw0, packed)
    pp0 = lax.bitcast_convert_type(p0, jnp.float32)
    xw1 = _relu_matmul_packed(pp0, jnp.tile(b0, 8), bd1)
    p1 = agg(xw1.reshape(n_nodes_p, d), packed)
    pp1 = lax.bitcast_convert_type(p1, jnp.float32)
    h1 = _relu_bias_packed(pp1, jnp.tile(b1, 8))
    p2 = agg(h1.reshape(n_nodes_p, d), packed)
    pp2 = lax.bitcast_convert_type(p2, jnp.float32)
    outp = _final_packed(pp2, bd2, b2)
    out = outp.reshape(n_nodes_p, 64)
    return out[:n_nodes]
